# baseline (speedup 1.0000x reference)
import functools

import jax
import jax.numpy as jnp
from jax import lax
from jax.experimental import pallas as pl
from jax.experimental.pallas import tpu as pltpu
from jax.experimental.pallas import tpu_sc as plsc

B = 16384
N_DIMS = 244
POLICY_DIM = 384
TOP_K = 32
N_CAT = 16
HALF = POLICY_DIM // 2

NW = 32
ROWS_W = B // NW
CH = 32
BLK = 2048


def _rev(k):
    return lax.rev(k, (0,))


def _merge16(Ak, Av, Bk, Bv):
    Brk, Brv = _rev(Bk), _rev(Bv)
    c = Ak >= Brk
    hk = jnp.where(c, Ak, Brk)
    hv = jnp.where(c, Av, Brv)
    lk = jnp.where(c, Brk, Ak)
    lv = jnp.where(c, Brv, Av)
    hk, hv = plsc.sort_key_val(hk, hv, descending=True)
    lk, lv = plsc.sort_key_val(lk, lv, descending=True)
    return (hk, hv), (lk, lv)


def _merge32_top(A, B):
    (A1k, A1v), (A2k, A2v) = A
    (B1k, B1v), (B2k, B2v) = B
    rB2k, rB2v = _rev(B2k), _rev(B2v)
    rB1k, rB1v = _rev(B1k), _rev(B1v)
    c1 = A1k >= rB2k
    T1k = jnp.where(c1, A1k, rB2k)
    T1v = jnp.where(c1, A1v, rB2v)
    c2 = A2k >= rB1k
    T2k = jnp.where(c2, A2k, rB1k)
    T2v = jnp.where(c2, A2v, rB1v)
    c = T1k >= T2k
    Uk = jnp.where(c, T1k, T2k)
    Uv = jnp.where(c, T1v, T2v)
    Lk = jnp.where(c, T2k, T1k)
    Lv = jnp.where(c, T2v, T1v)
    Uk, Uv = plsc.sort_key_val(Uk, Uv, descending=True)
    Lk, Lv = plsc.sort_key_val(Lk, Lv, descending=True)
    return (Uk, Uv), (Lk, Lv)


def _topk_features_sc(position, velocity):
    mesh = plsc.VectorSubcoreMesh(core_axis_name="c", subcore_axis_name="s")

    @functools.partial(
        pl.kernel,
        out_type=jax.ShapeDtypeStruct((B, 2 * TOP_K), jnp.float32),
        mesh=mesh,
        scratch_types=[
            pltpu.VMEM((CH, N_DIMS), jnp.float32),
            pltpu.VMEM((CH, N_DIMS), jnp.float32),
            pltpu.VMEM((CH, 2 * TOP_K), jnp.float32),
        ],
        compiler_params=pltpu.CompilerParams(needs_layout_passes=False),
    )
    def topk_kernel(pos_hbm, vel_hbm, feats_hbm, pos_v, vel_v, feats_v):
        cid = lax.axis_index("c")
        sid = lax.axis_index("s")
        wid = sid * 2 + cid
        base = wid * ROWS_W
        lanes = lax.iota(jnp.int32, 16)

        def chunk_body(ci, carry):
            cbase = base + ci * CH
            pltpu.sync_copy(pos_hbm.at[pl.ds(cbase, CH)], pos_v)
            pltpu.sync_copy(vel_hbm.at[pl.ds(cbase, CH)], vel_v)

            @plsc.parallel_loop(0, CH, 1, unroll=2)
            def row_body(r):
                leaves = []
                for j in range(15):
                    v = vel_v[r, pl.ds(j * 16, 16)]
                    leaves.append(plsc.sort_key_val(
                        jnp.abs(v), lanes + j * 16, descending=True))
                v = vel_v[r, pl.ds(N_DIMS - 16, 16)]
                k = jnp.where(lanes >= 12, jnp.abs(v), -1.0)
                leaves.append(plsc.sort_key_val(
                    k, lanes + (N_DIMS - 16), descending=True))

                l32 = [_merge16(*leaves[2 * i], *leaves[2 * i + 1])
                       for i in range(8)]
                l64 = [_merge32_top(l32[2 * i], l32[2 * i + 1])
                       for i in range(4)]
                l128 = [_merge32_top(l64[2 * i], l64[2 * i + 1])
                        for i in range(2)]
                (_, Uv), (_, Lv) = _merge32_top(l128[0], l128[1])

                rowv = jnp.full((16,), r, jnp.int32)
                pU = plsc.load_gather(pos_v, [rowv, Uv])
                pL = plsc.load_gather(pos_v, [rowv, Lv])
                vU = plsc.load_gather(vel_v, [rowv, Uv])
                vL = plsc.load_gather(vel_v, [rowv, Lv])
                feats_v[r, pl.ds(0, 16)] = pU
                feats_v[r, pl.ds(16, 16)] = pL
                feats_v[r, pl.ds(32, 16)] = vU
                feats_v[r, pl.ds(48, 16)] = vL

            pltpu.sync_copy(feats_v, feats_hbm.at[pl.ds(cbase, CH)])
            return carry

        lax.fori_loop(0, ROWS_W // CH, chunk_body, 0)

    return topk_kernel(position, velocity)


def _ln(x, g, b, eps=1e-5):
    m = jnp.mean(x, axis=-1, keepdims=True)
    v = jnp.mean((x - m) ** 2, axis=-1, keepdims=True)
    return (x - m) * lax.rsqrt(v + eps) * g + b


def _mlp_body(feats_ref, cat_ref, w1_ref, b1_ref, g1_ref, be1_ref,
              w2_ref, b2_ref, g2_ref, be2_ref, w3_ref, b3_ref, g3_ref,
              be3_ref, out_ref):
    h1 = lax.dot_general(feats_ref[...], w1_ref[...], (((1,), (1,)), ((), ())),
                         preferred_element_type=jnp.float32)
    h1 = jax.nn.relu(_ln(h1 + b1_ref[...], g1_ref[...], be1_ref[...]))
    h2 = lax.dot_general(cat_ref[...], w2_ref[...], (((1,), (1,)), ((), ())),
                         preferred_element_type=jnp.float32)
    h2 = jax.nn.relu(_ln(h2 + b2_ref[...], g2_ref[...], be2_ref[...]))
    fused = jnp.concatenate([h1, h2], axis=1).astype(jnp.bfloat16)
    h3 = lax.dot_general(fused, w3_ref[...].astype(jnp.bfloat16),
                         (((1,), (1,)), ((), ())),
                         preferred_element_type=jnp.float32)
    out_ref[...] = _ln(h3 + b3_ref[...], g3_ref[...], be3_ref[...])


@jax.jit
def kernel(position, velocity, categories, W1, b1, g1, be1, W2, b2, g2, be2,
           W3, b3, g3, be3):
    feats = _topk_features_sc(position, velocity)

    def rows(i):
        return (i, 0)

    def rep(i):
        return (0, 0)

    row_spec = lambda d: pl.BlockSpec((BLK, d), rows)
    full_spec = lambda s0, s1: pl.BlockSpec((s0, s1), rep)
    vec = lambda v: v.reshape(1, -1)

    return pl.pallas_call(
        _mlp_body,
        grid=(B // BLK,),
        in_specs=[
            row_spec(2 * TOP_K), row_spec(N_CAT),
            full_spec(HALF, 2 * TOP_K), full_spec(1, HALF), full_spec(1, HALF), full_spec(1, HALF),
            full_spec(HALF, N_CAT), full_spec(1, HALF), full_spec(1, HALF), full_spec(1, HALF),
            full_spec(POLICY_DIM, POLICY_DIM), full_spec(1, POLICY_DIM), full_spec(1, POLICY_DIM), full_spec(1, POLICY_DIM),
        ],
        out_specs=row_spec(POLICY_DIM),
        out_shape=jax.ShapeDtypeStruct((B, POLICY_DIM), jnp.float32),
        compiler_params=pltpu.CompilerParams(
            dimension_semantics=("arbitrary",),
        ),
    )(feats, categories,
      W1, vec(b1), vec(g1), vec(be1),
      W2, vec(b2), vec(g2), vec(be2),
      W3, vec(b3), vec(g3), vec(be3))


# speedup vs baseline: 1.0333x; 1.0333x over previous
import functools

import jax
import jax.numpy as jnp
from jax import lax
from jax.experimental import pallas as pl
from jax.experimental.pallas import tpu as pltpu
from jax.experimental.pallas import tpu_sc as plsc

B = 16384
N_DIMS = 244
POLICY_DIM = 384
TOP_K = 32
N_CAT = 16
HALF = POLICY_DIM // 2

NW = 32
ROWS_W = B // NW
CH = 32
BLK = 2048


def _rev(k):
    return lax.rev(k, (0,))


def _merge16(Ak, Av, Bk, Bv):
    Brk, Brv = _rev(Bk), _rev(Bv)
    c = Ak >= Brk
    hk = jnp.where(c, Ak, Brk)
    hv = jnp.where(c, Av, Brv)
    lk = jnp.where(c, Brk, Ak)
    lv = jnp.where(c, Brv, Av)
    hk, hv = plsc.sort_key_val(hk, hv, descending=True)
    lk, lv = plsc.sort_key_val(lk, lv, descending=True)
    return (hk, hv), (lk, lv)


def _merge32_top(A, B):
    (A1k, A1v), (A2k, A2v) = A
    (B1k, B1v), (B2k, B2v) = B
    rB2k, rB2v = _rev(B2k), _rev(B2v)
    rB1k, rB1v = _rev(B1k), _rev(B1v)
    c1 = A1k >= rB2k
    T1k = jnp.where(c1, A1k, rB2k)
    T1v = jnp.where(c1, A1v, rB2v)
    c2 = A2k >= rB1k
    T2k = jnp.where(c2, A2k, rB1k)
    T2v = jnp.where(c2, A2v, rB1v)
    c = T1k >= T2k
    Uk = jnp.where(c, T1k, T2k)
    Uv = jnp.where(c, T1v, T2v)
    Lk = jnp.where(c, T2k, T1k)
    Lv = jnp.where(c, T2v, T1v)
    Uk, Uv = plsc.sort_key_val(Uk, Uv, descending=True)
    Lk, Lv = plsc.sort_key_val(Lk, Lv, descending=True)
    return (Uk, Uv), (Lk, Lv)


def _topk_features_sc(position, velocity):
    mesh = plsc.VectorSubcoreMesh(core_axis_name="c", subcore_axis_name="s")

    @functools.partial(
        pl.kernel,
        out_type=jax.ShapeDtypeStruct((B, 2 * TOP_K), jnp.float32),
        mesh=mesh,
        scratch_types=[
            pltpu.VMEM((CH, N_DIMS), jnp.float32),
            pltpu.VMEM((CH, N_DIMS), jnp.float32),
            pltpu.VMEM((CH, 2 * TOP_K), jnp.float32),
        ],
        compiler_params=pltpu.CompilerParams(needs_layout_passes=False),
    )
    def topk_kernel(pos_hbm, vel_hbm, feats_hbm, pos_v, vel_v, feats_v):
        cid = lax.axis_index("c")
        sid = lax.axis_index("s")
        wid = sid * 2 + cid
        base = wid * ROWS_W
        lanes = lax.iota(jnp.int32, 16)

        def chunk_body(ci, carry):
            cbase = base + ci * CH
            pltpu.sync_copy(pos_hbm.at[pl.ds(cbase, CH)], pos_v)
            pltpu.sync_copy(vel_hbm.at[pl.ds(cbase, CH)], vel_v)

            @plsc.parallel_loop(0, CH, 1, unroll=2)
            def row_body(r):
                leaves = []
                for j in range(15):
                    v = vel_v[r, pl.ds(j * 16, 16)]
                    leaves.append(plsc.sort_key_val(
                        jnp.abs(v), lanes + j * 16, descending=True))
                v = vel_v[r, pl.ds(N_DIMS - 16, 16)]
                k = jnp.where(lanes >= 12, jnp.abs(v), -1.0)
                leaves.append(plsc.sort_key_val(
                    k, lanes + (N_DIMS - 16), descending=True))

                l32 = [_merge16(*leaves[2 * i], *leaves[2 * i + 1])
                       for i in range(8)]
                l64 = [_merge32_top(l32[2 * i], l32[2 * i + 1])
                       for i in range(4)]
                l128 = [_merge32_top(l64[2 * i], l64[2 * i + 1])
                        for i in range(2)]
                (_, Uv), (_, Lv) = _merge32_top(l128[0], l128[1])

                rowv = jnp.full((16,), r, jnp.int32)
                pU = plsc.load_gather(pos_v, [rowv, Uv])
                pL = plsc.load_gather(pos_v, [rowv, Lv])
                vU = plsc.load_gather(vel_v, [rowv, Uv])
                vL = plsc.load_gather(vel_v, [rowv, Lv])
                feats_v[r, pl.ds(0, 16)] = pU
                feats_v[r, pl.ds(16, 16)] = pL
                feats_v[r, pl.ds(32, 16)] = vU
                feats_v[r, pl.ds(48, 16)] = vL

            pltpu.sync_copy(feats_v, feats_hbm.at[pl.ds(cbase, CH)])
            return carry

        lax.fori_loop(0, ROWS_W // CH, chunk_body, 0)

    return topk_kernel(position, velocity)


def _ln(x, g, b, eps=1e-5):
    d = x.shape[-1]
    ones = jnp.ones((d, 1), jnp.float32)
    dims = (((1,), (0,)), ((), ()))
    m = lax.dot_general(x, ones, dims,
                        preferred_element_type=jnp.float32) * (1.0 / d)
    s2 = lax.dot_general(x * x, ones, dims,
                         preferred_element_type=jnp.float32) * (1.0 / d)
    v = s2 - m * m
    return (x - m) * lax.rsqrt(v + eps) * g + b


def _mlp_body(feats_ref, cat_ref, w1_ref, b1_ref, g1_ref, be1_ref,
              w2_ref, b2_ref, g2_ref, be2_ref, w3_ref, b3_ref, g3_ref,
              be3_ref, out_ref):
    h1 = lax.dot_general(feats_ref[...], w1_ref[...], (((1,), (1,)), ((), ())),
                         preferred_element_type=jnp.float32)
    h1 = jax.nn.relu(_ln(h1 + b1_ref[...], g1_ref[...], be1_ref[...]))
    h2 = lax.dot_general(cat_ref[...], w2_ref[...], (((1,), (1,)), ((), ())),
                         preferred_element_type=jnp.float32)
    h2 = jax.nn.relu(_ln(h2 + b2_ref[...], g2_ref[...], be2_ref[...]))
    fused = jnp.concatenate([h1, h2], axis=1)
    h3 = lax.dot_general(fused, w3_ref[...], (((1,), (1,)), ((), ())),
                         preferred_element_type=jnp.float32)
    out_ref[...] = _ln(h3 + b3_ref[...], g3_ref[...], be3_ref[...])


@jax.jit
def kernel(position, velocity, categories, W1, b1, g1, be1, W2, b2, g2, be2,
           W3, b3, g3, be3):
    feats = _topk_features_sc(position, velocity)

    def rows(i):
        return (i, 0)

    def rep(i):
        return (0, 0)

    row_spec = lambda d: pl.BlockSpec((BLK, d), rows)
    full_spec = lambda s0, s1: pl.BlockSpec((s0, s1), rep)
    vec = lambda v: v.reshape(1, -1)

    return pl.pallas_call(
        _mlp_body,
        grid=(B // BLK,),
        in_specs=[
            row_spec(2 * TOP_K), row_spec(N_CAT),
            full_spec(HALF, 2 * TOP_K), full_spec(1, HALF), full_spec(1, HALF), full_spec(1, HALF),
            full_spec(HALF, N_CAT), full_spec(1, HALF), full_spec(1, HALF), full_spec(1, HALF),
            full_spec(POLICY_DIM, POLICY_DIM), full_spec(1, POLICY_DIM), full_spec(1, POLICY_DIM), full_spec(1, POLICY_DIM),
        ],
        out_specs=row_spec(POLICY_DIM),
        out_shape=jax.ShapeDtypeStruct((B, POLICY_DIM), jnp.float32),
        compiler_params=pltpu.CompilerParams(
            dimension_semantics=("arbitrary",),
        ),
    )(feats, categories,
      W1, vec(b1), vec(g1), vec(be1),
      W2, vec(b2), vec(g2), vec(be2),
      W3, vec(b3), vec(g3), vec(be3))
